# baseline (speedup 1.0000x reference)
import functools

import jax
import jax.numpy as jnp
from jax.experimental import pallas as pl
from jax.experimental.pallas import tpu as pltpu


def _gcn_body(x_ref, a_ref, w_ref, o_ref, support_ref):
    @pl.when(pl.program_id(0) == 0)
    def _compute_support():
        support_ref[...] = jnp.dot(
            x_ref[...], w_ref[...], preferred_element_type=jnp.float32
        )

    o_ref[...] = jnp.dot(
        a_ref[...], support_ref[...], preferred_element_type=jnp.float32
    )


@jax.jit
def kernel(inputs, adjacency_matrix, W):
    n, d_in = inputs.shape
    d_out = W.shape[1]

    bm = 512 if n >= 512 else n

    return pl.pallas_call(
        _gcn_body,
        grid=(pl.cdiv(n, bm),),
        in_specs=[
            pl.BlockSpec((n, d_in), lambda i: (0, 0)),
            pl.BlockSpec((bm, n), lambda i: (i, 0)),
            pl.BlockSpec((d_in, d_out), lambda i: (0, 0)),
        ],
        out_specs=pl.BlockSpec((bm, d_out), lambda i: (i, 0)),
        out_shape=jax.ShapeDtypeStruct((n, d_out), jnp.float32),
        scratch_shapes=[pltpu.VMEM((n, d_out), jnp.float32)],
        compiler_params=pltpu.CompilerParams(
            dimension_semantics=("arbitrary",),
        ),
    )(inputs, adjacency_matrix, W)


# speedup vs baseline: 1.0085x; 1.0085x over previous
import jax
import jax.numpy as jnp
from jax.experimental import pallas as pl
from jax.experimental.pallas import tpu as pltpu


def _gcn_body(x_ref, a_ref, w_ref, o_ref):
    ax = jnp.dot(a_ref[...], x_ref[...], preferred_element_type=jnp.float32)
    o_ref[...] = jnp.dot(ax, w_ref[...], preferred_element_type=jnp.float32)


@jax.jit
def kernel(inputs, adjacency_matrix, W):
    n, d_in = inputs.shape
    d_out = W.shape[1]

    bm = 400 if n % 400 == 0 else n

    return pl.pallas_call(
        _gcn_body,
        grid=(n // bm,),
        in_specs=[
            pl.BlockSpec((n, d_in), lambda i: (0, 0)),
            pl.BlockSpec((bm, n), lambda i: (i, 0)),
            pl.BlockSpec((d_in, d_out), lambda i: (0, 0)),
        ],
        out_specs=pl.BlockSpec((bm, d_out), lambda i: (i, 0)),
        out_shape=jax.ShapeDtypeStruct((n, d_out), jnp.float32),
        compiler_params=pltpu.CompilerParams(
            dimension_semantics=("parallel",)
        ),
    )(inputs, adjacency_matrix, W)


# speedup vs baseline: 1.0099x; 1.0014x over previous
import jax
import jax.numpy as jnp
from jax.experimental import pallas as pl
from jax.experimental.pallas import tpu as pltpu


def _gcn_body(x_ref, a_ref, w_ref, o_ref, support_ref):
    @pl.when(pl.program_id(0) == 0)
    def _compute_support():
        support_ref[...] = jnp.dot(
            x_ref[...], w_ref[...], preferred_element_type=jnp.float32
        )

    o_ref[...] = jnp.dot(
        a_ref[...], support_ref[...], preferred_element_type=jnp.float32
    )


@jax.jit
def kernel(inputs, adjacency_matrix, W):
    n, d_in = inputs.shape
    d_out = W.shape[1]

    bm = 400 if n % 400 == 0 else n

    return pl.pallas_call(
        _gcn_body,
        grid=(n // bm,),
        in_specs=[
            pl.BlockSpec((n, d_in), lambda i: (0, 0)),
            pl.BlockSpec((bm, n), lambda i: (i, 0)),
            pl.BlockSpec((d_in, d_out), lambda i: (0, 0)),
        ],
        out_specs=pl.BlockSpec((bm, d_out), lambda i: (i, 0)),
        out_shape=jax.ShapeDtypeStruct((n, d_out), jnp.float32),
        scratch_shapes=[pltpu.VMEM((n, d_out), jnp.float32)],
        compiler_params=pltpu.CompilerParams(
            dimension_semantics=("arbitrary",)
        ),
    )(inputs, adjacency_matrix, W)
